# baseline (speedup 1.0000x reference)
import functools

import jax
import jax.numpy as jnp
import numpy as np
from jax import lax
from jax.experimental import pallas as pl
from jax.experimental.pallas import tpu as pltpu
from jax.experimental.pallas import tpu_sc as plsc

B, L = 4096, 50
V, D = 12 * 64 * 64, 256
V4 = V // 4
H = 32
HP = 128
NC, NS = 2, 16
NW = NC * NS
BPW = B // NW


def _proj_body(t0_ref, t1_ref, t2_ref, t3_ref, w_ref, out_ref):
    acc = jnp.dot(t0_ref[...], w_ref[0], preferred_element_type=jnp.float32)
    acc += jnp.dot(t1_ref[...], w_ref[1], preferred_element_type=jnp.float32)
    acc += jnp.dot(t2_ref[...], w_ref[2], preferred_element_type=jnp.float32)
    acc += jnp.dot(t3_ref[...], w_ref[3], preferred_element_type=jnp.float32)
    out_ref[...] = acc.astype(jnp.bfloat16)


def _project_table(table, w4):
    blk = 1024
    nblk = V4 // blk
    return pl.pallas_call(
        _proj_body,
        grid=(nblk,),
        in_specs=[
            pl.BlockSpec((blk, D), lambda i: (i, 0)),
            pl.BlockSpec((blk, D), lambda i: (i + nblk, 0)),
            pl.BlockSpec((blk, D), lambda i: (i + 2 * nblk, 0)),
            pl.BlockSpec((blk, D), lambda i: (i + 3 * nblk, 0)),
            pl.BlockSpec((4, D, HP), lambda i: (0, 0, 0)),
        ],
        out_specs=pl.BlockSpec((blk, HP), lambda i: (i, 0)),
        out_shape=jax.ShapeDtypeStruct((V4, HP), jnp.bfloat16),
    )(table, table, table, table, w4)


def _sc_body(t1, gidxw, x1b, gidx_v, buf0, buf1, acc, sem0, sem1):
    wid = lax.axis_index("s") * NC + lax.axis_index("c")
    pltpu.sync_copy(gidxw.at[wid], gidx_v)

    def _gather(l, buf, sem):
        pltpu.async_copy(t1.at[gidx_v.at[l]], buf, sem)

    def _wait(buf, sem):
        pltpu.make_async_copy(t1.at[gidx_v.at[0]], buf, sem).wait()

    def _accum(buf, first):
        def body(bi, c):
            for u in range(4):
                b = bi * 4 + u
                w = buf[b]
                va, vb = plsc.unpack(plsc.bitcast(w, jnp.bfloat16),
                                     format=plsc.PackFormat.INTERLEAVED)
                if first:
                    acc[b, 0] = va
                    acc[b, 1] = vb
                else:
                    plsc.addupdate(acc.at[b, 0], va)
                    plsc.addupdate(acc.at[b, 1], vb)
            return c
        lax.fori_loop(0, BPW // 4, body, 0)

    _gather(0, buf0, sem0)
    _wait(buf0, sem0)
    _gather(1, buf1, sem1)
    _accum(buf0, True)
    _wait(buf1, sem1)
    _gather(2, buf0, sem0)
    _accum(buf1, False)

    def loop_body(i, c):
        l2 = 2 * i
        _wait(buf0, sem0)
        _gather(l2 + 1, buf1, sem1)
        _accum(buf0, False)
        _wait(buf1, sem1)

        @pl.when(i < (L // 2) - 1)
        def _():
            _gather(l2 + 2, buf0, sem0)

        _accum(buf1, False)
        return c

    lax.fori_loop(1, L // 2, loop_body, 0)
    pltpu.sync_copy(acc, x1b.at[wid])


_sc_gather_sum = functools.partial(
    pl.kernel,
    out_type=jax.ShapeDtypeStruct((NW, BPW, 2, 16), jnp.float32),
    mesh=plsc.VectorSubcoreMesh(core_axis_name="c", subcore_axis_name="s"),
    compiler_params=pltpu.CompilerParams(use_tc_tiling_on_sc=False,
                                         needs_layout_passes=False),
    scratch_types=[
        pltpu.VMEM((L, BPW), jnp.int32),
        pltpu.VMEM((BPW, 16), jnp.int32),
        pltpu.VMEM((BPW, 16), jnp.int32),
        pltpu.VMEM((BPW, 2, 16), jnp.float32),
        pltpu.SemaphoreType.DMA,
        pltpu.SemaphoreType.DMA,
    ],
)(_sc_body)


def _mlp_body(x_ref, b1_ref, w2t_ref, b2_ref, w3t_ref, b3_ref, out_ref):
    h1 = jnp.maximum(x_ref[...] + b1_ref[...], 0.0)
    h2 = jnp.dot(h1, w2t_ref[...], preferred_element_type=jnp.float32)
    h2 = jnp.maximum(h2 + b2_ref[...], 0.0)
    out_ref[...] = (jnp.dot(h2, w3t_ref[...], preferred_element_type=jnp.float32)
                    + b3_ref[...])


def _mlp(x1, b1, w2t, b2, w3t, b3):
    return pl.pallas_call(
        _mlp_body,
        out_shape=jax.ShapeDtypeStruct((B, 1), jnp.float32),
    )(x1, b1.reshape(1, H), w2t, b2.reshape(1, H), w3t, b3.reshape(1, 1))


def kernel(indices, table, W1, b1, W2, b2, W3, b3):
    perm = np.array([2 * (j % 16) + j // 16 for j in range(H)])
    w4 = jnp.zeros((4, D, HP), jnp.float32)
    for r in range(4):
        w4 = w4.at[r, :, H * r + perm].set(W1)
    t1p = _project_table(table, w4)
    t1h = lax.bitcast_convert_type(t1p.reshape(V4, HP // 2, 2), jnp.int32)
    t1v = t1h.reshape(V, 16)
    idx = indices.astype(jnp.int32)
    gidxw = ((idx % V4) * 4 + idx // V4).reshape(NW, BPW, L).transpose(0, 2, 1)
    x1b = _sc_gather_sum(t1v, gidxw)
    out = _mlp(x1b.reshape(B, H), b1, W2.T, b2, W3.T, b3)
    return out[:, 0]


# speedup vs baseline: 2.0422x; 2.0422x over previous
import functools

import jax
import jax.numpy as jnp
from jax import lax
from jax.experimental import pallas as pl
from jax.experimental.pallas import tpu as pltpu
from jax.experimental.pallas import tpu_sc as plsc

B, L = 4096, 50
V, D = 12 * 64 * 64, 256
V4 = V // 4
H = 32
HP = 128
NC, NS = 2, 16
NW = NC * NS
BPW = B // NW


def _proj_body(t0_ref, t1_ref, t2_ref, t3_ref, w_ref, out_ref):
    acc = jnp.dot(t0_ref[...], w_ref[0], preferred_element_type=jnp.float32)
    acc += jnp.dot(t1_ref[...], w_ref[1], preferred_element_type=jnp.float32)
    acc += jnp.dot(t2_ref[...], w_ref[2], preferred_element_type=jnp.float32)
    acc += jnp.dot(t3_ref[...], w_ref[3], preferred_element_type=jnp.float32)
    out_ref[...] = acc


def _project_table(table, w4):
    blk = 1024
    nblk = V4 // blk
    return pl.pallas_call(
        _proj_body,
        grid=(nblk,),
        in_specs=[
            pl.BlockSpec((blk, D), lambda i: (i, 0)),
            pl.BlockSpec((blk, D), lambda i: (i + nblk, 0)),
            pl.BlockSpec((blk, D), lambda i: (i + 2 * nblk, 0)),
            pl.BlockSpec((blk, D), lambda i: (i + 3 * nblk, 0)),
            pl.BlockSpec((4, D, HP), lambda i: (0, 0, 0)),
        ],
        out_specs=pl.BlockSpec((blk, HP), lambda i: (i, 0)),
        out_shape=jax.ShapeDtypeStruct((V4, HP), jnp.float32),
    )(table, table, table, table, w4)


def _sc_body(t1, gidxw, x1b, gidx_v, buf0, buf1, buf2, buf3, acc,
             sem0, sem1, sem2, sem3):
    wid = lax.axis_index("s") * NC + lax.axis_index("c")
    pltpu.sync_copy(gidxw.at[wid], gidx_v)
    bufs = (buf0, buf1, buf2, buf3)
    sems = (sem0, sem1, sem2, sem3)

    def _gather(l, p):
        pltpu.async_copy(t1.at[gidx_v.at[l]], bufs[p], sems[p])

    def _wait(p):
        pltpu.make_async_copy(t1.at[gidx_v.at[0]], bufs[p], sems[p]).wait()

    def _accum(buf, first):
        def body(bi, c):
            for u in range(4):
                b = bi * 4 + u
                for h in range(2):
                    v = buf[b, pl.ds(16 * h, 16)]
                    if first:
                        acc[b, h] = v
                    else:
                        plsc.addupdate(acc.at[b, h], v)
            return c
        lax.fori_loop(0, BPW // 4, body, 0)

    for l in range(3):
        _gather(l, l)
    for p in range(4):
        _wait(p)
        _gather(p + 3, (p + 3) % 4)
        _accum(bufs[p], p == 0)

    def loop_body(i, c):
        for p in range(4):
            l = 4 * i + p
            _wait(p)

            @pl.when(l + 3 < L)
            def _():
                _gather(l + 3, (p + 3) % 4)

            _accum(bufs[p], False)
        return c

    lax.fori_loop(1, L // 4, loop_body, 0)
    for p in range(L % 4):
        _wait(p)
        _accum(bufs[p], False)

    pltpu.sync_copy(acc, x1b.at[wid])


_sc_gather_sum = functools.partial(
    pl.kernel,
    out_type=jax.ShapeDtypeStruct((NW, BPW, 2, 16), jnp.float32),
    mesh=plsc.VectorSubcoreMesh(core_axis_name="c", subcore_axis_name="s"),
    compiler_params=pltpu.CompilerParams(use_tc_tiling_on_sc=False),
    scratch_types=[
        pltpu.VMEM((L, BPW), jnp.int32),
        pltpu.VMEM((BPW, H), jnp.float32),
        pltpu.VMEM((BPW, H), jnp.float32),
        pltpu.VMEM((BPW, H), jnp.float32),
        pltpu.VMEM((BPW, H), jnp.float32),
        pltpu.VMEM((BPW, 2, 16), jnp.float32),
        pltpu.SemaphoreType.DMA,
        pltpu.SemaphoreType.DMA,
        pltpu.SemaphoreType.DMA,
        pltpu.SemaphoreType.DMA,
    ],
)(_sc_body)


def _mlp_body(x_ref, b1_ref, w2t_ref, b2_ref, w3t_ref, b3_ref, out_ref):
    h1 = jnp.maximum(x_ref[...] + b1_ref[...], 0.0)
    h2 = jnp.dot(h1, w2t_ref[...], preferred_element_type=jnp.float32)
    h2 = jnp.maximum(h2 + b2_ref[...], 0.0)
    out_ref[...] = (jnp.dot(h2, w3t_ref[...], preferred_element_type=jnp.float32)
                    + b3_ref[...])


def _mlp(x1, b1, w2t, b2, w3t, b3):
    return pl.pallas_call(
        _mlp_body,
        out_shape=jax.ShapeDtypeStruct((B, 1), jnp.float32),
    )(x1, b1.reshape(1, H), w2t, b2.reshape(1, H), w3t, b3.reshape(1, 1))


def kernel(indices, table, W1, b1, W2, b2, W3, b3):
    w4 = jnp.zeros((4, D, HP), jnp.float32)
    for r in range(4):
        w4 = w4.at[r, :, H * r:H * (r + 1)].set(W1.T)
    t1p = _project_table(table, w4)
    t1v = t1p.reshape(V, H)
    idx = indices.astype(jnp.int32)
    gidxw = ((idx % V4) * 4 + idx // V4).reshape(NW, BPW, L).transpose(0, 2, 1)
    x1b = _sc_gather_sum(t1v, gidxw)
    out = _mlp(x1b.reshape(B, H), b1, W2.T, b2, W3.T, b3)
    return out[:, 0]


# speedup vs baseline: 2.0982x; 1.0275x over previous
import functools

import jax
import jax.numpy as jnp
from jax import lax
from jax.experimental import pallas as pl
from jax.experimental.pallas import tpu as pltpu
from jax.experimental.pallas import tpu_sc as plsc

B, L = 4096, 50
V, D = 12 * 64 * 64, 256
V4 = V // 4
H = 32
HP = 128
NC, NS = 2, 16
NW = NC * NS
BPW = B // NW


def _proj_body(t0_ref, t1_ref, t2_ref, t3_ref, w_ref, out_ref):
    acc = jnp.dot(t0_ref[...], w_ref[0], preferred_element_type=jnp.float32)
    acc += jnp.dot(t1_ref[...], w_ref[1], preferred_element_type=jnp.float32)
    acc += jnp.dot(t2_ref[...], w_ref[2], preferred_element_type=jnp.float32)
    acc += jnp.dot(t3_ref[...], w_ref[3], preferred_element_type=jnp.float32)
    out_ref[...] = acc


def _project_table(table, w4):
    blk = 2048
    nblk = V4 // blk
    return pl.pallas_call(
        _proj_body,
        grid=(nblk,),
        in_specs=[
            pl.BlockSpec((blk, D), lambda i: (i, 0)),
            pl.BlockSpec((blk, D), lambda i: (i + nblk, 0)),
            pl.BlockSpec((blk, D), lambda i: (i + 2 * nblk, 0)),
            pl.BlockSpec((blk, D), lambda i: (i + 3 * nblk, 0)),
            pl.BlockSpec((4, D, HP), lambda i: (0, 0, 0)),
        ],
        out_specs=pl.BlockSpec((blk, HP), lambda i: (i, 0)),
        out_shape=jax.ShapeDtypeStruct((V4, HP), jnp.float32),
    )(table, table, table, table, w4)


RB = 5


def _sc_body(t1, gidxw, x1b, gidx_v, buf0, buf1, buf2, buf3, buf4, acc,
             sem0, sem1, sem2, sem3, sem4):
    wid = lax.axis_index("s") * NC + lax.axis_index("c")
    pltpu.sync_copy(gidxw.at[wid], gidx_v)
    bufs = (buf0, buf1, buf2, buf3, buf4)
    sems = (sem0, sem1, sem2, sem3, sem4)

    def _gather(l, p):
        pltpu.async_copy(t1.at[gidx_v.at[l]], bufs[p], sems[p])

    def _wait(p):
        pltpu.make_async_copy(t1.at[gidx_v.at[0]], bufs[p], sems[p]).wait()

    def _accum(buf, first):
        def body(bi, c):
            for u in range(8):
                b = bi * 8 + u
                for h in range(2):
                    v = buf[b, pl.ds(16 * h, 16)]
                    if first:
                        acc[b, h] = v
                    else:
                        plsc.addupdate(acc.at[b, h], v)
            return c
        lax.fori_loop(0, BPW // 8, body, 0)

    for l in range(RB - 1):
        _gather(l, l)
    for p in range(RB):
        _wait(p)
        _gather(p + RB - 1, (p + RB - 1) % RB)
        _accum(bufs[p], p == 0)

    def loop_body(i, c):
        for p in range(RB):
            l = RB * i + p
            _wait(p)

            @pl.when(l + RB - 1 < L)
            def _():
                _gather(l + RB - 1, (p + RB - 1) % RB)

            _accum(bufs[p], False)
        return c

    lax.fori_loop(1, L // RB, loop_body, 0)

    pltpu.sync_copy(acc, x1b.at[wid])


_sc_gather_sum = functools.partial(
    pl.kernel,
    out_type=jax.ShapeDtypeStruct((NW, BPW, 2, 16), jnp.float32),
    mesh=plsc.VectorSubcoreMesh(core_axis_name="c", subcore_axis_name="s"),
    compiler_params=pltpu.CompilerParams(use_tc_tiling_on_sc=False),
    scratch_types=[
        pltpu.VMEM((L, BPW), jnp.int32),
        pltpu.VMEM((BPW, H), jnp.float32),
        pltpu.VMEM((BPW, H), jnp.float32),
        pltpu.VMEM((BPW, H), jnp.float32),
        pltpu.VMEM((BPW, H), jnp.float32),
        pltpu.VMEM((BPW, H), jnp.float32),
        pltpu.VMEM((BPW, 2, 16), jnp.float32),
        pltpu.SemaphoreType.DMA,
        pltpu.SemaphoreType.DMA,
        pltpu.SemaphoreType.DMA,
        pltpu.SemaphoreType.DMA,
        pltpu.SemaphoreType.DMA,
    ],
)(_sc_body)


def _mlp_body(x_ref, b1_ref, w2t_ref, b2_ref, w3t_ref, b3_ref, out_ref):
    h1 = jnp.maximum(x_ref[...] + b1_ref[...], 0.0)
    h2 = jnp.dot(h1, w2t_ref[...], preferred_element_type=jnp.float32)
    h2 = jnp.maximum(h2 + b2_ref[...], 0.0)
    out_ref[...] = (jnp.dot(h2, w3t_ref[...], preferred_element_type=jnp.float32)
                    + b3_ref[...])


def _mlp(x1, b1, w2t, b2, w3t, b3):
    return pl.pallas_call(
        _mlp_body,
        out_shape=jax.ShapeDtypeStruct((B, 1), jnp.float32),
    )(x1, b1.reshape(1, H), w2t, b2.reshape(1, H), w3t, b3.reshape(1, 1))


def kernel(indices, table, W1, b1, W2, b2, W3, b3):
    w4 = jnp.zeros((4, D, HP), jnp.float32)
    for r in range(4):
        w4 = w4.at[r, :, H * r:H * (r + 1)].set(W1.T)
    t1p = _project_table(table, w4)
    t1v = t1p.reshape(V, H)
    idx = indices.astype(jnp.int32)
    gidxw = ((idx % V4) * 4 + idx // V4).reshape(NW, BPW, L).transpose(0, 2, 1)
    x1b = _sc_gather_sum(t1v, gidxw)
    out = _mlp(x1b.reshape(B, H), b1, W2.T, b2, W3.T, b3)
    return out[:, 0]
